# baseline (speedup 1.0000x reference)
import jax
import jax.numpy as jnp
from jax.experimental import pallas as pl

_K, _PAD, _STRIDE = 5, 1, 2
_H = 128
_OH = (_H + 2 * _PAD - _K) // _STRIDE + 1
_OWP = 64
_L64 = _OH * _OWP
_C25 = 64 * _K * _K
_LP = 4096
_CP = 1664
_TI = 1024
_TJ = 2048


def _patch_matrix(x):
    xc = x[0]
    xp = jnp.pad(xc, ((0, 0), (_PAD, _PAD), (_PAD, _PAD)))
    ph = [[xp[:, a::2, b::2] for b in range(2)] for a in range(2)]
    bj = [[jnp.pad(ph[a][j % 2][:, :, j // 2:j // 2 + _OH],
                   ((0, 0), (0, 0), (0, _OWP - _OH)), mode="edge")
           for j in range(_K)] for a in range(2)]
    rows = []
    for i in range(_K):
        for j in range(_K):
            rows.append(bj[i % 2][j][:, i // 2:i // 2 + _OH, :])
    p = jnp.stack(rows, axis=0)
    m = p.reshape(_C25, _L64)
    m = jnp.pad(m, ((0, 0), (0, _LP - _L64)), mode="edge")
    return jnp.pad(m, ((0, _CP - _C25), (0, 0)))


def _body(k_ref, q_ref, o_ref):
    i = pl.program_id(1)
    kb = k_ref[...].astype(jnp.float32)
    inv = jax.lax.rsqrt(jnp.maximum(jnp.sum(kb * kb, axis=0, keepdims=True), 1e-24))
    kn = (kb * inv).astype(jnp.bfloat16)
    r = jax.lax.dot_general(
        kn, q_ref[...],
        dimension_numbers=(((0,), (0,)), ((), ())),
        preferred_element_type=jnp.float32,
    )
    m = jnp.max(r, axis=0, keepdims=True)
    acc = jnp.where(i == 0, jnp.full_like(m, -jnp.inf), o_ref[...])
    o_ref[...] = jnp.maximum(acc, m)

    @pl.when(i == pl.num_programs(1) - 1)
    def _():
        qf = q_ref[...].astype(jnp.float32)
        qn = jnp.sqrt(jnp.sum(qf * qf, axis=0, keepdims=True))
        o_ref[...] = o_ref[...] / jnp.maximum(qn, 1e-12)


def kernel(feat_edit, feat_ori, feat_2d):
    del feat_2d
    k_bf = _patch_matrix(feat_edit).astype(jnp.bfloat16)
    q_bf = _patch_matrix(feat_ori).astype(jnp.bfloat16)

    out = pl.pallas_call(
        _body,
        grid=(_LP // _TJ, _LP // _TI),
        in_specs=[
            pl.BlockSpec((_CP, _TI), lambda j, i: (0, i)),
            pl.BlockSpec((_CP, _TJ), lambda j, i: (0, j)),
        ],
        out_specs=pl.BlockSpec((1, _TJ), lambda j, i: (0, j)),
        out_shape=jax.ShapeDtypeStruct((1, _LP), jnp.float32),
    )(k_bf, q_bf)

    return out[0, :_L64].reshape(_OH, _OWP)[:, :_OH].reshape(1, 1, _OH, _OH)


# speedup vs baseline: 1.1969x; 1.1969x over previous
import jax
import jax.numpy as jnp
from jax.experimental import pallas as pl

_K, _PAD = 5, 1
_OH = 63
_OWP = 64
_L64 = _OH * _OWP
_C25 = 64 * _K * _K
_CP = 1664
_LP = 4096
_W = 512
_NW = _LP // _W
_TI = 1024
_TJ = 2048
_NKT = _LP // _TI
_NQT = _LP // _TJ


def _shift_views(x):
    xp = jnp.pad(x[0], ((0, 0), (_PAD, _PAD), (_PAD, _PAD)))
    ph = [[xp[:, a::2, b::2] for b in range(2)] for a in range(2)]
    imgs = []
    for a in range(2):
        for j in range(_K):
            im = ph[a][j % 2][:, :, j // 2:j // 2 + _OH]
            im = jnp.pad(im, ((0, 0), (0, 0), (0, _OWP - _OH)), mode="edge")
            im = jnp.pad(im, ((0, 0), (0, 1), (0, 0)))
            imgs.append(im)
    b = jnp.stack(imgs, axis=0).astype(jnp.bfloat16)
    flat = b.reshape(10, 64, 66 * _OWP)
    return [flat[:, :, v * _OWP:v * _OWP + _LP] for v in range(3)]


def _gather(b0, b1, b2):
    by_off = (b0, b1, b2)
    parts = []
    for i in range(_K):
        for j in range(_K):
            parts.append(by_off[i // 2][(i % 2) * _K + j, :, :])
    parts.append(jnp.zeros((_CP - _C25, parts[0].shape[1]), jnp.bfloat16))
    return jnp.concatenate(parts, axis=0)


def _im2col_body(k0, k1, k2, q0, q1, q2, kn_ref, qm_ref):
    ct = pl.program_id(0)
    kc = _gather(k0[...], k1[...], k2[...]).astype(jnp.float32)
    inv = jax.lax.rsqrt(jnp.maximum(jnp.sum(kc * kc, axis=0, keepdims=True), 1e-24))
    kn_ref[...] = (kc * inv).astype(jnp.bfloat16)
    qm_ref[...] = _gather(q0[...], q1[...], q2[...])

    @pl.when(ct == _NW - 1)
    def _():
        lo = _L64 - (_NW - 1) * _W
        kn_ref[:, lo:_W] = kn_ref[:, lo - _OWP:_W - _OWP]


def _sim_body(kn_ref, qm_ref, o_ref):
    i = pl.program_id(1)
    r = jax.lax.dot_general(
        kn_ref[...], qm_ref[...],
        dimension_numbers=(((0,), (0,)), ((), ())),
        preferred_element_type=jnp.float32,
    )
    m = jnp.max(r, axis=0, keepdims=True)
    acc = jnp.where(i == 0, jnp.full_like(m, -jnp.inf), o_ref[...])
    o_ref[...] = jnp.maximum(acc, m)

    @pl.when(i == pl.num_programs(1) - 1)
    def _():
        qf = qm_ref[...].astype(jnp.float32)
        qn = jnp.sqrt(jnp.sum(qf * qf, axis=0, keepdims=True))
        o_ref[...] = o_ref[...] / jnp.maximum(qn, 1e-12)


def kernel(feat_edit, feat_ori, feat_2d):
    del feat_2d
    kv = _shift_views(feat_edit)
    qv = _shift_views(feat_ori)

    bspec = pl.BlockSpec((10, 64, _W), lambda ct: (0, 0, ct))
    kn, qm = pl.pallas_call(
        _im2col_body,
        grid=(_NW,),
        in_specs=[bspec] * 6,
        out_specs=[pl.BlockSpec((_CP, _W), lambda ct: (0, ct))] * 2,
        out_shape=[jax.ShapeDtypeStruct((_CP, _LP), jnp.bfloat16)] * 2,
    )(*kv, *qv)

    out = pl.pallas_call(
        _sim_body,
        grid=(_NQT, _NKT),
        in_specs=[
            pl.BlockSpec((_CP, _TI), lambda j, i: (0, i)),
            pl.BlockSpec((_CP, _TJ), lambda j, i: (0, j)),
        ],
        out_specs=pl.BlockSpec((1, _TJ), lambda j, i: (0, j)),
        out_shape=jax.ShapeDtypeStruct((1, _LP), jnp.float32),
    )(kn, qm)

    return out[0, :_L64].reshape(_OH, _OWP)[:, :_OH].reshape(1, 1, _OH, _OH)


# speedup vs baseline: 1.7749x; 1.4830x over previous
import jax
import jax.numpy as jnp
from jax.experimental import pallas as pl

_K, _PAD = 5, 1
_OH = 63
_OWP = 64
_L64 = _OH * _OWP
_C25 = 64 * _K * _K
_CP = 1664
_LP = 4096
_YP = 72
_FW = _YP * _OWP
_W = 512
_NW = _LP // _W
_TI = 1024
_TJ = 2048
_NKT = _LP // _TI
_NQT = _LP // _TJ


def _packed_phases(x):
    xb = jnp.pad(x[0], ((0, 0), (_PAD, _PAD), (_PAD, _PAD))).astype(jnp.bfloat16)
    b32 = jax.lax.bitcast_convert_type(xb.reshape(64, 130, 65, 2), jnp.uint32)
    return b32[:, 0::2, :], b32[:, 1::2, :]


def _emit_shift_images(e_ref, o_ref, out_ref):
    zrow = jnp.zeros((64, _YP - 65, _OWP), jnp.bfloat16)
    for a, ref in ((0, e_ref), (1, o_ref)):
        va = ref[...]
        xe = jax.lax.bitcast_convert_type(va << 16, jnp.float32)
        xo = jax.lax.bitcast_convert_type(va & jnp.uint32(0xFFFF0000),
                                          jnp.float32)
        ph = (xe, xo)
        for j in range(_K):
            im = ph[j % 2][:, :, j // 2:j // 2 + _OH]
            im = jnp.concatenate([im, im[:, :, _OH - 1:_OH]], axis=2)
            im16 = jnp.concatenate([im.astype(jnp.bfloat16), zrow], axis=1)
            out_ref[a * _K + j] = im16


def _prep_body(ke, ko, qe, qo, fk_ref, fq_ref):
    _emit_shift_images(ke, ko, fk_ref)
    _emit_shift_images(qe, qo, fq_ref)


def _window(b1, b2, s, d):
    if d == 0:
        return b1[s, :, :]
    return jnp.concatenate([b1[s, :, 64 * d:], b2[s, :, :64 * d]], axis=1)


def _gather(b1, b2):
    parts = []
    for i in range(_K):
        for j in range(_K):
            parts.append(_window(b1, b2, (i % 2) * _K + j, i // 2))
    parts.append(jnp.zeros((_CP - _C25, _W), jnp.bfloat16))
    return jnp.concatenate(parts, axis=0)


def _im2col_body(k1, k2, q1, q2, kn_ref, qm_ref):
    ct = pl.program_id(0)
    kc = _gather(k1[...], k2[...]).astype(jnp.float32)
    inv = jax.lax.rsqrt(jnp.maximum(jnp.sum(kc * kc, axis=0, keepdims=True), 1e-24))
    kn_ref[...] = (kc * inv).astype(jnp.bfloat16)
    qm_ref[...] = _gather(q1[...], q2[...])

    @pl.when(ct == _NW - 1)
    def _():
        lo = _L64 - (_NW - 1) * _W
        kn_ref[:, lo:_W] = kn_ref[:, lo - _OWP:_W - _OWP]


def _sim_body(kn_ref, qm_ref, o_ref):
    i = pl.program_id(1)
    r = jax.lax.dot_general(
        kn_ref[...], qm_ref[...],
        dimension_numbers=(((0,), (0,)), ((), ())),
        preferred_element_type=jnp.float32,
    )
    m = jnp.max(r, axis=0, keepdims=True)
    acc = jnp.where(i == 0, jnp.full_like(m, -jnp.inf), o_ref[...])
    o_ref[...] = jnp.maximum(acc, m)

    @pl.when(i == pl.num_programs(1) - 1)
    def _():
        qf = qm_ref[...].astype(jnp.float32)
        qn = jnp.sqrt(jnp.sum(qf * qf, axis=0, keepdims=True))
        o_ref[...] = o_ref[...] / jnp.maximum(qn, 1e-12)


def kernel(feat_edit, feat_ori, feat_2d):
    del feat_2d
    ke, ko = _packed_phases(feat_edit)
    qe, qo = _packed_phases(feat_ori)

    pspec = pl.BlockSpec((64, 65, 65), lambda: (0, 0, 0))
    fk, fq = pl.pallas_call(
        _prep_body,
        grid=(),
        in_specs=[pspec] * 4,
        out_specs=[pl.BlockSpec((10, 64, _YP, _OWP), lambda: (0, 0, 0, 0))] * 2,
        out_shape=[jax.ShapeDtypeStruct((10, 64, _YP, _OWP), jnp.bfloat16)] * 2,
    )(ke, ko, qe, qo)

    fk = fk.reshape(10, 64, _FW)
    fq = fq.reshape(10, 64, _FW)

    spec1 = pl.BlockSpec((10, 64, _W), lambda ct: (0, 0, ct))
    spec2 = pl.BlockSpec((10, 64, _W), lambda ct: (0, 0, ct + 1))
    kn, qm = pl.pallas_call(
        _im2col_body,
        grid=(_NW,),
        in_specs=[spec1, spec2, spec1, spec2],
        out_specs=[pl.BlockSpec((_CP, _W), lambda ct: (0, ct))] * 2,
        out_shape=[jax.ShapeDtypeStruct((_CP, _LP), jnp.bfloat16)] * 2,
    )(fk, fk, fq, fq)

    out = pl.pallas_call(
        _sim_body,
        grid=(_NQT, _NKT),
        in_specs=[
            pl.BlockSpec((_CP, _TI), lambda j, i: (0, i)),
            pl.BlockSpec((_CP, _TJ), lambda j, i: (0, j)),
        ],
        out_specs=pl.BlockSpec((1, _TJ), lambda j, i: (0, j)),
        out_shape=jax.ShapeDtypeStruct((1, _LP), jnp.float32),
    )(kn, qm)

    return out[0, :_L64].reshape(_OH, _OWP)[:, :_OH].reshape(1, 1, _OH, _OH)


# speedup vs baseline: 3.4358x; 1.9358x over previous
import jax
import jax.numpy as jnp
from jax.experimental import pallas as pl
from jax.experimental.pallas import tpu as pltpu

_K = 5
_OH = 63
_OWP = 64
_L64 = _OH * _OWP
_C25 = 64 * _K * _K
_CP = 1664
_LP = 4096
_YP = 96
_FW = _YP * _OWP
_TI = 1024
_TJ = 1024
_NKT = _LP // _TI
_NQT = _LP // _TJ


def _pack_pairs(x):
    xb = x[0].astype(jnp.bfloat16)
    b32 = jax.lax.bitcast_convert_type(xb.reshape(64, 128, 64, 2), jnp.uint32)
    return b32.reshape(64, 64, 128)


def _emit_shift_images(v_ref, out_ref):
    v = v_ref[...]
    zrow_u = jnp.zeros((64, 1, 64), jnp.uint32)
    zcol = jnp.zeros((64, 65, 1), jnp.float32)
    zpad = jnp.zeros((64, _YP - 65, _OWP), jnp.bfloat16)
    for a in range(2):
        if a == 0:
            va = jnp.concatenate([zrow_u, v[:, :, 64:]], axis=1)
        else:
            va = jnp.concatenate([v[:, :, :64], zrow_u], axis=1)
        hi = jax.lax.bitcast_convert_type(va & jnp.uint32(0xFFFF0000),
                                          jnp.float32)
        lo = jax.lax.bitcast_convert_type(va << 16, jnp.float32)
        ph = (jnp.concatenate([zcol, hi], axis=2),
              jnp.concatenate([lo, zcol], axis=2))
        for j in range(_K):
            im = ph[j % 2][:, :, j // 2:j // 2 + _OH]
            im = jnp.concatenate([im, im[:, :, _OH - 1:_OH]], axis=2)
            out_ref[a * _K + j] = jnp.concatenate(
                [im.astype(jnp.bfloat16), zpad], axis=1)


def _prep_body(vk_ref, vq_ref, fk_ref, fq_ref):
    _emit_shift_images(vk_ref, fk_ref)
    _emit_shift_images(vq_ref, fq_ref)


def _window(b1, b2, s, d, w):
    if d == 0:
        return b1[s, :, :]
    return jnp.concatenate([b1[s, :, 64 * d:], b2[s, :, :64 * d]], axis=1)


def _gather(b1, b2, w):
    parts = []
    for i in range(_K):
        for j in range(_K):
            parts.append(_window(b1, b2, (i % 2) * _K + j, i // 2, w))
    parts.append(jnp.zeros((_CP - _C25, w), jnp.bfloat16))
    return jnp.concatenate(parts, axis=0)


def _sim_body(k1, k2, q1, q2, o_ref, kn_ref, qm_ref):
    j = pl.program_id(0)
    i = pl.program_id(1)

    @pl.when(j == 0)
    def _():
        kc = _gather(k1[...], k2[...], _TI).astype(jnp.float32)
        inv = jax.lax.rsqrt(
            jnp.maximum(jnp.sum(kc * kc, axis=0, keepdims=True), 1e-24))
        kn_ref[i] = (kc * inv).astype(jnp.bfloat16)

        @pl.when(i == _NKT - 1)
        def _():
            lo = _L64 - (_NKT - 1) * _TI
            kn_ref[_NKT - 1, :, lo:_TI] = kn_ref[_NKT - 1, :, lo - _OWP:_TI - _OWP]

    @pl.when(i == 0)
    def _():
        qm_ref[...] = _gather(q1[...], q2[...], _TJ)

    r = jax.lax.dot_general(
        kn_ref[i], qm_ref[...],
        dimension_numbers=(((0,), (0,)), ((), ())),
        preferred_element_type=jnp.float32,
    )
    m = jnp.max(r, axis=0, keepdims=True)
    acc = jnp.where(i == 0, jnp.full_like(m, -jnp.inf), o_ref[...])
    o_ref[...] = jnp.maximum(acc, m)

    @pl.when(i == pl.num_programs(1) - 1)
    def _():
        qf = qm_ref[...].astype(jnp.float32)
        qn = jnp.sqrt(jnp.sum(qf * qf, axis=0, keepdims=True))
        o_ref[...] = o_ref[...] / jnp.maximum(qn, 1e-12)


def kernel(feat_edit, feat_ori, feat_2d):
    del feat_2d
    vk = _pack_pairs(feat_edit)
    vq = _pack_pairs(feat_ori)

    pspec = pl.BlockSpec((64, 64, 128), lambda: (0, 0, 0))
    fk, fq = pl.pallas_call(
        _prep_body,
        grid=(),
        in_specs=[pspec] * 2,
        out_specs=[pl.BlockSpec((10, 64, _YP, _OWP), lambda: (0, 0, 0, 0))] * 2,
        out_shape=[jax.ShapeDtypeStruct((10, 64, _YP, _OWP), jnp.bfloat16)] * 2,
    )(vk, vq)

    fk = fk.reshape(10, 64, _FW)
    fq = fq.reshape(10, 64, _FW)

    kspec1 = pl.BlockSpec((10, 64, _TI), lambda j, i: (0, 0, i))
    kspec2 = pl.BlockSpec((10, 64, _TI), lambda j, i: (0, 0, i + 1))
    qspec1 = pl.BlockSpec((10, 64, _TJ), lambda j, i: (0, 0, j))
    qspec2 = pl.BlockSpec((10, 64, _TJ), lambda j, i: (0, 0, j + 1))
    out = pl.pallas_call(
        _sim_body,
        grid=(_NQT, _NKT),
        in_specs=[kspec1, kspec2, qspec1, qspec2],
        out_specs=pl.BlockSpec((1, _TJ), lambda j, i: (0, j)),
        out_shape=jax.ShapeDtypeStruct((1, _LP), jnp.float32),
        scratch_shapes=[
            pltpu.VMEM((_NKT, _CP, _TI), jnp.bfloat16),
            pltpu.VMEM((_CP, _TJ), jnp.bfloat16),
        ],
    )(fk, fk, fq, fq)

    return out[0, :_L64].reshape(_OH, _OWP)[:, :_OH].reshape(1, 1, _OH, _OH)


# speedup vs baseline: 3.4817x; 1.0134x over previous
import jax
import jax.numpy as jnp
from jax.experimental import pallas as pl
from jax.experimental.pallas import tpu as pltpu

_K = 5
_OH = 63
_OWP = 64
_L64 = _OH * _OWP
_C25 = 64 * _K * _K
_CP = 1664
_LP = 4096
_YP = 96
_FW = _YP * _OWP
_TI = 1024
_TJ = 2048
_NKT = _LP // _TI
_NQT = _LP // _TJ
_QR = _TJ // _OWP


def _pack_pairs(x):
    xb = x[0].astype(jnp.bfloat16)
    b32 = jax.lax.bitcast_convert_type(xb.reshape(64, 128, 64, 2), jnp.uint32)
    return b32.reshape(64, 64, 128)


def _emit_shift_images(v_ref, out_ref):
    v = v_ref[...]
    zrow_u = jnp.zeros((64, 1, 64), jnp.uint32)
    zcol = jnp.zeros((64, 65, 1), jnp.float32)
    zpad = jnp.zeros((64, _YP - 65, _OWP), jnp.bfloat16)
    for a in range(2):
        if a == 0:
            va = jnp.concatenate([zrow_u, v[:, :, 64:]], axis=1)
        else:
            va = jnp.concatenate([v[:, :, :64], zrow_u], axis=1)
        hi = jax.lax.bitcast_convert_type(va & jnp.uint32(0xFFFF0000),
                                          jnp.float32)
        lo = jax.lax.bitcast_convert_type(va << 16, jnp.float32)
        ph = (jnp.concatenate([zcol, hi], axis=2),
              jnp.concatenate([lo, zcol], axis=2))
        for j in range(_K):
            im = ph[j % 2][:, :, j // 2:j // 2 + _OH]
            im = jnp.concatenate([im, im[:, :, _OH - 1:_OH]], axis=2)
            out_ref[a * _K + j] = jnp.concatenate(
                [im.astype(jnp.bfloat16), zpad], axis=1)


def _prep_body(vk_ref, vq_ref, fk_ref, fq_ref):
    _emit_shift_images(vk_ref, fk_ref)
    _emit_shift_images(vq_ref, fq_ref)


def _window(b1, b2, s, d, w):
    if d == 0:
        return b1[s, :, :]
    return jnp.concatenate([b1[s, :, 64 * d:], b2[s, :, :64 * d]], axis=1)


def _gather(b1, b2, w):
    parts = []
    for i in range(_K):
        for j in range(_K):
            parts.append(_window(b1, b2, (i % 2) * _K + j, i // 2, w))
    parts.append(jnp.zeros((_CP - _C25, w), jnp.bfloat16))
    return jnp.concatenate(parts, axis=0)


def _sim_body(k1, k2, q1, q2, o_ref, kn_ref, qm_ref, acc_ref):
    j = pl.program_id(0)
    i = pl.program_id(1)

    @pl.when(j == 0)
    def _():
        kc = _gather(k1[...], k2[...], _TI).astype(jnp.float32)
        inv = jax.lax.rsqrt(
            jnp.maximum(jnp.sum(kc * kc, axis=0, keepdims=True), 1e-24))
        kn_ref[i] = (kc * inv).astype(jnp.bfloat16)

        @pl.when(i == _NKT - 1)
        def _():
            lo = _L64 - (_NKT - 1) * _TI
            kn_ref[_NKT - 1, :, lo:_TI] = kn_ref[_NKT - 1, :, lo - _OWP:_TI - _OWP]

    @pl.when(i == 0)
    def _():
        qm_ref[...] = _gather(q1[...], q2[...], _TJ)

    r = jax.lax.dot_general(
        kn_ref[i], qm_ref[...],
        dimension_numbers=(((0,), (0,)), ((), ())),
        preferred_element_type=jnp.float32,
    )
    m = jnp.max(r, axis=0, keepdims=True)
    acc_ref[...] = jnp.where(i == 0, m, jnp.maximum(acc_ref[...], m))

    @pl.when(i == pl.num_programs(1) - 1)
    def _():
        qf = qm_ref[...].astype(jnp.float32)
        qn = jnp.sqrt(jnp.sum(qf * qf, axis=0, keepdims=True))
        fin = acc_ref[...] / jnp.maximum(qn, 1e-12)
        v = jnp.concatenate(
            [fin[:, k * _OWP:(k + 1) * _OWP] for k in range(_QR)], axis=0)
        for jj in range(_NQT):
            r0 = jj * _QR
            nr = min(_QR, _OH - r0)

            @pl.when(j == jj)
            def _():
                o_ref[0, 0, r0:r0 + nr, :] = v[:nr, :_OH]


def kernel(feat_edit, feat_ori, feat_2d):
    del feat_2d
    vk = _pack_pairs(feat_edit)
    vq = _pack_pairs(feat_ori)

    pspec = pl.BlockSpec((64, 64, 128), lambda: (0, 0, 0))
    fk, fq = pl.pallas_call(
        _prep_body,
        grid=(),
        in_specs=[pspec] * 2,
        out_specs=[pl.BlockSpec((10, 64, _YP, _OWP), lambda: (0, 0, 0, 0))] * 2,
        out_shape=[jax.ShapeDtypeStruct((10, 64, _YP, _OWP), jnp.bfloat16)] * 2,
    )(vk, vq)

    fk = fk.reshape(10, 64, _FW)
    fq = fq.reshape(10, 64, _FW)

    kspec1 = pl.BlockSpec((10, 64, _TI), lambda j, i: (0, 0, i))
    kspec2 = pl.BlockSpec((10, 64, _TI), lambda j, i: (0, 0, i + 1))
    qspec1 = pl.BlockSpec((10, 64, _TJ), lambda j, i: (0, 0, j))
    qspec2 = pl.BlockSpec((10, 64, _TJ), lambda j, i: (0, 0, j + 1))
    return pl.pallas_call(
        _sim_body,
        grid=(_NQT, _NKT),
        in_specs=[kspec1, kspec2, qspec1, qspec2],
        out_specs=pl.BlockSpec((1, 1, _OH, _OH), lambda j, i: (0, 0, 0, 0)),
        out_shape=jax.ShapeDtypeStruct((1, 1, _OH, _OH), jnp.float32),
        scratch_shapes=[
            pltpu.VMEM((_NKT, _CP, _TI), jnp.bfloat16),
            pltpu.VMEM((_CP, _TJ), jnp.bfloat16),
            pltpu.VMEM((1, _TJ), jnp.float32),
        ],
    )(fk, fk, fq, fq)
